# baseline (speedup 1.0000x reference)
import functools

import jax
import jax.numpy as jnp
from jax import lax
from jax.experimental import pallas as pl
from jax.experimental.pallas import tpu as pltpu
from jax.experimental.pallas import tpu_sc as plsc

_CHUNK = 128


@functools.lru_cache(maxsize=None)
def _make_sc_gather(V, D, B):
    info = plsc.get_sparse_core_info()
    NC, NS = info.num_cores, info.num_subcores
    NW = NC * NS
    assert B % NW == 0
    b_per_w = B // NW
    assert b_per_w % _CHUNK == 0
    n_chunks = b_per_w // _CHUNK
    mesh = plsc.VectorSubcoreMesh(core_axis_name="c", subcore_axis_name="s")

    @functools.partial(
        pl.kernel,
        mesh=mesh,
        compiler_params=pltpu.CompilerParams(use_tc_tiling_on_sc=False),
        out_type=jax.ShapeDtypeStruct((B, D), jnp.float32),
        scratch_types=[
            pltpu.VMEM((b_per_w,), jnp.int32),
            pltpu.VMEM((b_per_w, D), jnp.float32),
            pltpu.SemaphoreType.DMA,
        ],
    )
    def sc_gather(table_hbm, idx_hbm, out_hbm, idx_v, rows_v, sem):
        wid = lax.axis_index("s") * NC + lax.axis_index("c")
        base = wid * b_per_w
        pltpu.sync_copy(idx_hbm.at[pl.ds(base, b_per_w)], idx_v)
        copies = []
        for j in range(n_chunks):
            copies.append(
                pltpu.async_copy(
                    table_hbm.at[idx_v.at[pl.ds(j * _CHUNK, _CHUNK)]],
                    rows_v.at[pl.ds(j * _CHUNK, _CHUNK)],
                    sem,
                )
            )
        for c in copies:
            c.wait()
        pltpu.sync_copy(rows_v, out_hbm.at[pl.ds(base, b_per_w)])

    return sc_gather


def _mlp_block(x_ref, w1_ref, b1_ref, g_ref, be_ref, mu_ref, var_ref,
               w2_ref, b2_ref, o_ref):
    x = x_ref[...]
    h = jnp.dot(x, w1_ref[...], preferred_element_type=jnp.float32)
    h = jnp.maximum(h + b1_ref[...], 0.0)
    s = g_ref[...] * lax.rsqrt(var_ref[...] + 1e-5)
    t = be_ref[...] - mu_ref[...] * s
    h = h * s + t
    o = jnp.dot(h, w2_ref[...], preferred_element_type=jnp.float32)
    o_ref[...] = jnp.maximum(o + b2_ref[...], 0.0)


def _mlp(x, W1, b1, gamma, beta, mu, var, W2, b2, block_b):
    B, D = x.shape
    H = W1.shape[1]
    grid = (B // block_b,)
    row = lambda v: v.reshape(1, H)
    rep = lambda shape: pl.BlockSpec(shape, lambda i: (0, 0))
    return pl.pallas_call(
        _mlp_block,
        grid=grid,
        in_specs=[
            pl.BlockSpec((block_b, D), lambda i: (i, 0)),
            rep((D, H)), rep((1, H)), rep((1, H)), rep((1, H)),
            rep((1, H)), rep((1, H)), rep((H, H)), rep((1, H)),
        ],
        out_specs=pl.BlockSpec((block_b, H), lambda i: (i, 0)),
        out_shape=jax.ShapeDtypeStruct((B, H), jnp.float32),
    )(x, W1, row(b1), row(gamma), row(beta), row(mu), row(var), W2, row(b2))


def kernel(target_ids, emb_table, W1, b1, gamma, beta, running_mean,
           running_var, W2, b2):
    V, D = emb_table.shape
    B = target_ids.shape[0]
    ids = target_ids.astype(jnp.int32)
    x = _make_sc_gather(V, D, B)(emb_table, ids)
    return _mlp(x, W1, b1, gamma, beta, running_mean, running_var, W2, b2,
                block_b=2048)


# speedup vs baseline: 2.2691x; 2.2691x over previous
import functools

import jax
import jax.numpy as jnp
from jax import lax
from jax.experimental import pallas as pl
from jax.experimental.pallas import tpu as pltpu
from jax.experimental.pallas import tpu_sc as plsc

_SLICE_W = 128
_NBUF = 8


@functools.lru_cache(maxsize=None)
def _make_sc_gather(V, D, B):
    info = plsc.get_sparse_core_info()
    NC, NS = info.num_cores, info.num_subcores
    NW = NC * NS
    assert B % (16 * NW) == 0
    b_per_w = B // NW
    n_groups = b_per_w // 16
    mesh = plsc.VectorSubcoreMesh(core_axis_name="c", subcore_axis_name="s")

    @functools.partial(
        pl.kernel,
        mesh=mesh,
        compiler_params=pltpu.CompilerParams(needs_layout_passes=False),
        out_type=jax.ShapeDtypeStruct((B * D,), jnp.float32),
        scratch_types=[
            pltpu.VMEM((b_per_w,), jnp.int32),
            pltpu.VMEM((_NBUF, D, _SLICE_W), jnp.float32),
            pltpu.VMEM((16 * D,), jnp.float32),
            pltpu.SemaphoreType.DMA,
        ],
    )
    def sc_gather(tableT_hbm, idx_hbm, out_hbm, idx_v, stage_v, rows_v, sem):
        wid = lax.axis_index("s") * NC + lax.axis_index("c")
        base = wid * b_per_w
        pltpu.sync_copy(idx_hbm.at[pl.ds(base, b_per_w)], idx_v)
        iota16 = lax.broadcasted_iota(jnp.int32, (16,), 0)

        def group(g, carry):
            vvec = idx_v[pl.ds(g * 16, 16)]
            for half in range(2):
                for b in range(8):
                    v = vvec[half * 8 + b]
                    start = pl.multiple_of((v // _SLICE_W) * _SLICE_W, 128)
                    pltpu.async_copy(
                        tableT_hbm.at[pl.ds(0, D), pl.ds(start, _SLICE_W)],
                        stage_v.at[b], sem,
                    )
                for b in range(8):
                    pltpu.make_async_copy(
                        tableT_hbm.at[pl.ds(0, D), pl.ds(0, _SLICE_W)],
                        stage_v.at[b], sem,
                    ).wait()
                for b in range(8):
                    v = vvec[half * 8 + b]
                    l = v - (v // _SLICE_W) * _SLICE_W
                    lanes = iota16 * 0 + l
                    for k in range(D // 16):
                        rows16 = iota16 + k * 16
                        col = plsc.load_gather(stage_v.at[b], [rows16, lanes])
                        rows_v[pl.ds((half * 8 + b) * D + k * 16, 16)] = col
            pltpu.sync_copy(
                rows_v, out_hbm.at[pl.ds((base + g * 16) * D, 16 * D)]
            )
            return carry

        lax.fori_loop(0, n_groups, group, 0)

    return sc_gather


def _mlp_block(x_ref, w1_ref, b1_ref, g_ref, be_ref, mu_ref, var_ref,
               w2_ref, b2_ref, o_ref):
    x = x_ref[...]
    h = jnp.dot(x, w1_ref[...], preferred_element_type=jnp.float32)
    h = jnp.maximum(h + b1_ref[...], 0.0)
    s = g_ref[...] * lax.rsqrt(var_ref[...] + 1e-5)
    t = be_ref[...] - mu_ref[...] * s
    h = h * s + t
    o = jnp.dot(h, w2_ref[...], preferred_element_type=jnp.float32)
    o_ref[...] = jnp.maximum(o + b2_ref[...], 0.0)


def _mlp(x, W1, b1, gamma, beta, mu, var, W2, b2, block_b):
    B, D = x.shape
    H = W1.shape[1]
    grid = (B // block_b,)
    row = lambda v: v.reshape(1, H)
    rep = lambda shape: pl.BlockSpec(shape, lambda i: (0, 0))
    return pl.pallas_call(
        _mlp_block,
        grid=grid,
        in_specs=[
            pl.BlockSpec((block_b, D), lambda i: (i, 0)),
            rep((D, H)), rep((1, H)), rep((1, H)), rep((1, H)),
            rep((1, H)), rep((1, H)), rep((H, H)), rep((1, H)),
        ],
        out_specs=pl.BlockSpec((block_b, H), lambda i: (i, 0)),
        out_shape=jax.ShapeDtypeStruct((B, H), jnp.float32),
    )(x, W1, row(b1), row(gamma), row(beta), row(mu), row(var), W2, row(b2))


def kernel(target_ids, emb_table, W1, b1, gamma, beta, running_mean,
           running_var, W2, b2):
    V, D = emb_table.shape
    B = target_ids.shape[0]
    ids = target_ids.astype(jnp.int32)
    x = _make_sc_gather(V, D, B)(emb_table.T, ids).reshape(B, D)
    return _mlp(x, W1, b1, gamma, beta, running_mean, running_var, W2, b2,
                block_b=2048)


# speedup vs baseline: 2.8114x; 1.2390x over previous
import functools

import jax
import jax.numpy as jnp
from jax import lax
from jax.experimental import pallas as pl
from jax.experimental.pallas import tpu as pltpu
from jax.experimental.pallas import tpu_sc as plsc

_SLICE_W = 128
_NBUF = 8


@functools.lru_cache(maxsize=None)
def _make_sc_gather(V, D, B):
    info = plsc.get_sparse_core_info()
    NC, NS = info.num_cores, info.num_subcores
    NW = NC * NS
    assert B % (16 * NW) == 0
    b_per_w = B // NW
    n_groups = b_per_w // 16
    mesh = plsc.VectorSubcoreMesh(core_axis_name="c", subcore_axis_name="s")

    @functools.partial(
        pl.kernel,
        mesh=mesh,
        compiler_params=pltpu.CompilerParams(needs_layout_passes=False),
        out_type=jax.ShapeDtypeStruct((B * D,), jnp.float32),
        scratch_types=[
            pltpu.VMEM((b_per_w + 16,), jnp.int32),
            pltpu.VMEM((_NBUF, D, _SLICE_W), jnp.float32),
            pltpu.VMEM((b_per_w * D,), jnp.float32),
        ] + [pltpu.SemaphoreType.DMA] * _NBUF,
    )
    def sc_gather(tableT_hbm, idx_hbm, out_hbm, idx_v, stage_v, rows_v, *sems):
        wid = lax.axis_index("s") * NC + lax.axis_index("c")
        base = wid * b_per_w
        pltpu.sync_copy(idx_hbm.at[pl.ds(base, b_per_w)],
                        idx_v.at[pl.ds(0, b_per_w)])
        iota16 = lax.broadcasted_iota(jnp.int32, (16,), 0)
        n_iter = b_per_w // _NBUF

        def issue(v, b):
            start = pl.multiple_of((v // _SLICE_W) * _SLICE_W, 128)
            pltpu.async_copy(
                tableT_hbm.at[pl.ds(0, D), pl.ds(start, _SLICE_W)],
                stage_v.at[b], sems[b],
            )

        vvec0 = idx_v[pl.ds(0, 16)]
        for b in range(_NBUF):
            issue(vvec0[b], b)

        def step(g, carry):
            vvec = idx_v[pl.ds(g * _NBUF, 16)]
            for b in range(_NBUF):
                pltpu.make_async_copy(
                    tableT_hbm.at[pl.ds(0, D), pl.ds(0, _SLICE_W)],
                    stage_v.at[b], sems[b],
                ).wait()
                v = vvec[b]
                l = v - (v // _SLICE_W) * _SLICE_W
                lanes = iota16 * 0 + l
                for k in range(D // 16):
                    rows16 = iota16 + k * 16
                    col = plsc.load_gather(stage_v.at[b], [rows16, lanes])
                    rows_v[pl.ds(g * _NBUF * D + b * D + k * 16, 16)] = col

                @pl.when(g < n_iter - 1)
                def _():
                    issue(vvec[_NBUF + b], b)

            return carry

        lax.fori_loop(0, n_iter, step, 0)
        pltpu.sync_copy(rows_v, out_hbm.at[pl.ds(base * D, b_per_w * D)])

    return sc_gather


def _mlp_block(x_ref, w1_ref, b1_ref, g_ref, be_ref, mu_ref, var_ref,
               w2_ref, b2_ref, o_ref):
    x = x_ref[...]
    h = jnp.dot(x, w1_ref[...], preferred_element_type=jnp.float32)
    h = jnp.maximum(h + b1_ref[...], 0.0)
    s = g_ref[...] * lax.rsqrt(var_ref[...] + 1e-5)
    t = be_ref[...] - mu_ref[...] * s
    h = h * s + t
    o = jnp.dot(h, w2_ref[...], preferred_element_type=jnp.float32)
    o_ref[...] = jnp.maximum(o + b2_ref[...], 0.0)


def _mlp(x, W1, b1, gamma, beta, mu, var, W2, b2, block_b):
    B, D = x.shape
    H = W1.shape[1]
    grid = (B // block_b,)
    row = lambda v: v.reshape(1, H)
    rep = lambda shape: pl.BlockSpec(shape, lambda i: (0, 0))
    return pl.pallas_call(
        _mlp_block,
        grid=grid,
        in_specs=[
            pl.BlockSpec((block_b, D), lambda i: (i, 0)),
            rep((D, H)), rep((1, H)), rep((1, H)), rep((1, H)),
            rep((1, H)), rep((1, H)), rep((H, H)), rep((1, H)),
        ],
        out_specs=pl.BlockSpec((block_b, H), lambda i: (i, 0)),
        out_shape=jax.ShapeDtypeStruct((B, H), jnp.float32),
    )(x, W1, row(b1), row(gamma), row(beta), row(mu), row(var), W2, row(b2))


def kernel(target_ids, emb_table, W1, b1, gamma, beta, running_mean,
           running_var, W2, b2):
    V, D = emb_table.shape
    B = target_ids.shape[0]
    ids = target_ids.astype(jnp.int32)
    x = _make_sc_gather(V, D, B)(emb_table.T, ids).reshape(B, D)
    return _mlp(x, W1, b1, gamma, beta, running_mean, running_var, W2, b2,
                block_b=2048)
